# baseline (speedup 1.0000x reference)
import jax
import jax.numpy as jnp
from jax import lax
from jax.experimental import pallas as pl
from jax.experimental.pallas import tpu as pltpu

B, L, NL, NF, NV = 16, 128, 40, 32, 20
T = B * NV
NBUF = 8


def _fpred_body(lf_ref, vlab_ref, out_ref):
    def body(i, carry):
        oh = (lax.broadcasted_iota(jnp.int32, (NV, L), 1)
              == vlab_ref[i]).astype(jnp.float32)
        fp = jnp.dot(oh, lf_ref[i],
                     preferred_element_type=jnp.float32)
        out_ref[i] = fp.reshape(NV, NF, 1)
        return carry

    lax.fori_loop(0, B, body, 0)


def _loss_body(vlab_ref, fidx_ref, vl_ref, lp_hbm, pool_hbm, fp_ref, out_ref,
               lp_buf, pool_buf, sem_lp, sem_pool):
    def issue(p):
        slot = lax.rem(p, NBUF)
        pltpu.make_async_copy(
            lp_hbm.at[p // NV, vlab_ref[p]], lp_buf.at[slot],
            sem_lp.at[slot]).start()
        pltpu.make_async_copy(
            pool_hbm.at[fidx_ref[p]], pool_buf.at[slot],
            sem_pool.at[slot]).start()

    lax.fori_loop(0, NBUF, lambda p, c: (issue(p), c)[1], 0)

    def step(p, acc):
        slot = lax.rem(p, NBUF)
        pltpu.make_async_copy(lp_hbm.at[0, 0], lp_buf.at[slot],
                              sem_lp.at[slot]).wait()
        pltpu.make_async_copy(pool_hbm.at[0], pool_buf.at[slot],
                              sem_pool.at[slot]).wait()
        x = lp_buf[slot]
        xmax = jnp.max(x, axis=0, keepdims=True)
        w = jnp.maximum(1.0 - jnp.exp(xmax), 1e-6)
        cand = jnp.where(pool_buf[slot] == 0, w, 1.0)
        wm = jnp.min(cand, axis=1, keepdims=True)
        m = jnp.log(wm)
        s = jnp.sum(jnp.maximum(fp_ref[p] - m, 0.0))

        @pl.when(p + NBUF < T)
        def _next():
            issue(p + NBUF)

        maskf = jnp.where(lax.rem(p, NV) < vl_ref[p // NV], 1.0, 0.0)
        return acc + maskf * s

    total = lax.fori_loop(0, T, step, jnp.float32(0.0))
    tot = lax.fori_loop(0, B, lambda i, a: a + vl_ref[i], 0)
    norm = jnp.maximum(tot, 1).astype(jnp.float32)
    out_ref[...] = jnp.full((1, 1), total / norm, jnp.float32)


@jax.jit
def _frame_role_loss(log_pa, v_label, v_l, log_frame, frame_idx, frame_pool):
    vlab = v_label.astype(jnp.int32)
    vlab_flat = vlab.reshape(-1)
    fidx = jnp.take_along_axis(frame_idx.astype(jnp.int32), vlab, axis=1)
    fidx_flat = fidx.reshape(-1)
    vl = v_l.astype(jnp.int32)

    fpred = pl.pallas_call(
        _fpred_body,
        in_specs=[
            pl.BlockSpec((B, L, NF), lambda: (0, 0, 0)),
            pl.BlockSpec((B, NV, 1), lambda: (0, 0, 0)),
        ],
        out_shape=jax.ShapeDtypeStruct((B, NV, NF, 1), jnp.float32),
        out_specs=pl.BlockSpec((B, NV, NF, 1), lambda: (0, 0, 0, 0)),
    )(log_frame, vlab.reshape(B, NV, 1))

    out = pl.pallas_call(
        _loss_body,
        in_specs=[
            pl.BlockSpec(memory_space=pltpu.SMEM),
            pl.BlockSpec(memory_space=pltpu.SMEM),
            pl.BlockSpec(memory_space=pltpu.SMEM),
            pl.BlockSpec(memory_space=pl.ANY),
            pl.BlockSpec(memory_space=pl.ANY),
            pl.BlockSpec((T, NF, 1), lambda: (0, 0, 0)),
        ],
        out_shape=jax.ShapeDtypeStruct((1, 1), jnp.float32),
        scratch_shapes=[
            pltpu.VMEM((NBUF, L, NL), jnp.float32),
            pltpu.VMEM((NBUF, NF, NL), jnp.int32),
            pltpu.SemaphoreType.DMA((NBUF,)),
            pltpu.SemaphoreType.DMA((NBUF,)),
        ],
    )(vlab_flat, fidx_flat, vl, log_pa, frame_pool,
      fpred.reshape(T, NF, 1))
    return out.reshape(())


def kernel(log_pa, score, v_label, v_l, role_label, roleset_id, log_frame,
           frame_idx, frame_pool):
    return _frame_role_loss(log_pa, v_label, v_l, log_frame, frame_idx,
                            frame_pool)


# speedup vs baseline: 1.2412x; 1.2412x over previous
import functools

import jax
import jax.numpy as jnp
from jax import lax
from jax.experimental import pallas as pl
from jax.experimental.pallas import tpu as pltpu

B, L, NL, NF, NV = 16, 128, 40, 32, 20
NW = 32
RPW = (B * NV) // NW
NEGBIG = -1.0e30


def _fpred_body(lf_ref, vlab_ref, vl_ref, out_ref):
    def body(i, carry):
        oh = (lax.broadcasted_iota(jnp.int32, (NV, L), 1)
              == vlab_ref[i]).astype(jnp.float32)
        fp = jnp.dot(oh, lf_ref[i],
                     preferred_element_type=jnp.float32)
        mask = lax.broadcasted_iota(jnp.int32, (NV, 1), 0) < vl_ref[i]
        out_ref[i] = jnp.where(mask, fp, NEGBIG)
        return carry

    lax.fori_loop(0, B, body, 0)


def _loss_body(vlab_ref, fidx_ref, vl_ref, *refs):
    lp_refs = refs[:RPW]
    pool_refs = refs[RPW:2 * RPW]
    fp_ref = refs[2 * RPW]
    out_ref = refs[2 * RPW + 1]
    g = pl.program_id(0)

    @pl.when(g == 0)
    def _init():
        out_ref[...] = jnp.zeros((1, 1), jnp.float32)

    x10 = jnp.concatenate([r[0] for r in lp_refs], axis=0)
    xmax = jnp.max(x10, axis=1)
    w = jnp.maximum(1.0 - jnp.exp(xmax), 1e-6)
    pool10 = jnp.concatenate([r[...] for r in pool_refs],
                             axis=0)
    cand = jnp.where(pool10 == 0, w[:, None, :], 1.0)
    wm = jnp.min(cand, axis=2)
    m = jnp.log(wm)
    s = jnp.sum(jnp.maximum(fp_ref[0] - m, 0.0))
    out_ref[...] += jnp.full((1, 1), s, jnp.float32)

    @pl.when(g == NW - 1)
    def _fini():
        tot = lax.fori_loop(0, B, lambda i, a: a + vl_ref[i], 0)
        norm = jnp.maximum(tot, 1).astype(jnp.float32)
        out_ref[...] = out_ref[...] / norm


@jax.jit
def _frame_role_loss(log_pa, v_label, v_l, log_frame, frame_idx, frame_pool):
    vlab = v_label.astype(jnp.int32)
    vlab_flat = vlab.reshape(-1)
    fidx = jnp.take_along_axis(frame_idx.astype(jnp.int32), vlab, axis=1)
    fidx_flat = fidx.reshape(-1)
    vl = v_l.astype(jnp.int32)

    fpred = pl.pallas_call(
        _fpred_body,
        in_specs=[
            pl.BlockSpec((B, L, NF), lambda: (0, 0, 0)),
            pl.BlockSpec((B, NV, 1), lambda: (0, 0, 0)),
            pl.BlockSpec((B, 1), lambda: (0, 0)),
        ],
        out_shape=jax.ShapeDtypeStruct((B, NV, NF), jnp.float32),
        out_specs=pl.BlockSpec((B, NV, NF), lambda: (0, 0, 0)),
    )(log_frame, vlab.reshape(B, NV, 1), vl.reshape(B, 1))

    lp_specs = [
        pl.BlockSpec((1, 1, L, NL),
                     functools.partial(
                         lambda k, g, vlab, fidx, vl:
                         ((g * RPW + k) // NV, vlab[g * RPW + k], 0, 0), k))
        for k in range(RPW)
    ]
    pool_specs = [
        pl.BlockSpec((1, NF, NL),
                     functools.partial(
                         lambda k, g, vlab, fidx, vl:
                         (fidx[g * RPW + k], 0, 0), k))
        for k in range(RPW)
    ]
    fp_spec = pl.BlockSpec((1, RPW, NF),
                           lambda g, vlab, fidx, vl: (g, 0, 0))

    grid_spec = pltpu.PrefetchScalarGridSpec(
        num_scalar_prefetch=3,
        grid=(NW,),
        in_specs=lp_specs + pool_specs + [fp_spec],
        out_specs=pl.BlockSpec((1, 1), lambda g, vlab, fidx, vl: (0, 0)),
    )
    out = pl.pallas_call(
        _loss_body,
        grid_spec=grid_spec,
        out_shape=jax.ShapeDtypeStruct((1, 1), jnp.float32),
    )(vlab_flat, fidx_flat, vl,
      *([log_pa] * RPW), *([frame_pool] * RPW), fpred.reshape(NW, RPW, NF))
    return out.reshape(())


def kernel(log_pa, score, v_label, v_l, role_label, roleset_id, log_frame,
           frame_idx, frame_pool):
    return _frame_role_loss(log_pa, v_label, v_l, log_frame, frame_idx,
                            frame_pool)
